# baseline (speedup 1.0000x reference)
import functools

import jax
import jax.numpy as jnp
from jax import lax
from jax.experimental import pallas as pl
from jax.experimental.pallas import tpu as pltpu
from jax.experimental.pallas import tpu_sc as plsc

NB = 1024
S = 50
D = 1000
NSUB = 8
QB = S * NSUB
NC, NS_ = 2, 16
NW = NC * NS_
BPW = NB // NW

_mesh = plsc.VectorSubcoreMesh(core_axis_name="c", subcore_axis_name="s")


@functools.partial(
    pl.kernel,
    mesh=_mesh,
    out_type=jax.ShapeDtypeStruct((NB, S, D), jnp.float32),
    scratch_types=[
        pltpu.VMEM((BPW * QB,), jnp.int32),
        pltpu.VMEM((QB, 128), jnp.float32),
        pltpu.VMEM((S, D), jnp.float32),
        pltpu.SemaphoreType.DMA,
    ],
)
def _gather_kernel(idx_hbm, w8_hbm, out_hbm, idx_v, stage_v, slab_v, sem):
    wid = lax.axis_index("s") * NC + lax.axis_index("c")
    b0 = wid * BPW
    pltpu.sync_copy(idx_hbm.at[pl.ds(b0 * QB, BPW * QB)], idx_v)

    def batch_body(k, carry):
        q0 = k * QB
        cps = []
        for j, n in ((0, 128), (128, 128), (256, 128), (384, 16)):
            cps.append(pltpu.async_copy(
                w8_hbm.at[idx_v.at[pl.ds(q0 + j, n)]],
                stage_v.at[pl.ds(j, n)], sem))
        for cp in cps:
            cp.wait()

        def row_body(s, c2):
            for t in range(NSUB - 1):
                for c in range(8):
                    slab_v[s, pl.ds(t * 128 + c * 16, 16)] = (
                        stage_v[s * NSUB + t, pl.ds(c * 16, 16)])
            for c in range(6):
                slab_v[s, pl.ds(896 + c * 16, 16)] = (
                    stage_v[s * NSUB + 7, pl.ds(c * 16, 16)])
            slab_v[s, pl.ds(D - 16, 16)] = (
                stage_v[s * NSUB + 7, pl.ds(104 - 16, 16)])
            return c2

        lax.fori_loop(0, S, row_body, 0)

        pltpu.sync_copy(slab_v, out_hbm.at[b0 + k])
        return carry

    lax.fori_loop(0, BPW, batch_body, 0)


def kernel(idx, w):
    w8 = jnp.pad(w, ((0, 0), (0, 24))).reshape(NSUB * 1000, 128)
    idx8 = (idx.astype(jnp.int32)[:, :, None] * NSUB
            + jnp.arange(NSUB, dtype=jnp.int32)).reshape(-1)
    return _gather_kernel(idx8, w8)


# speedup vs baseline: 1.8945x; 1.8945x over previous
import functools

import jax
import jax.numpy as jnp
from jax import lax
from jax.experimental import pallas as pl
from jax.experimental.pallas import tpu as pltpu
from jax.experimental.pallas import tpu_sc as plsc

NB = 1024
S = 50
D = 1000
NSUB = 8
QB = S * NSUB
TAIL = D - 7 * 128
NC, NS_ = 2, 16
NW = NC * NS_
BPW = NB // NW

_mesh = plsc.VectorSubcoreMesh(core_axis_name="c", subcore_axis_name="s")


@functools.partial(
    pl.kernel,
    mesh=_mesh,
    out_type=jax.ShapeDtypeStruct((NB, S, D), jnp.float32),
    scratch_types=[
        pltpu.VMEM((BPW * QB,), jnp.int32),
        pltpu.VMEM((QB, 128), jnp.float32),
        pltpu.VMEM((S, TAIL), jnp.float32),
        pltpu.SemaphoreType.DMA,
        pltpu.SemaphoreType.DMA,
    ],
)
def _gather_kernel(idx_hbm, w8_hbm, out_hbm, idx_v, stage_v, slab_v,
                   gsem, wsem):
    wid = lax.axis_index("s") * NC + lax.axis_index("c")
    b0 = wid * BPW
    pltpu.sync_copy(idx_hbm.at[pl.ds(b0 * QB, BPW * QB)], idx_v)

    def batch_body(k, carry):
        q0 = k * QB
        cps = []
        for j, n in ((0, 128), (128, 128), (256, 128), (384, 16)):
            cps.append(pltpu.async_copy(
                w8_hbm.at[idx_v.at[pl.ds(q0 + j, n)]],
                stage_v.at[pl.ds(j, n)], gsem))
        for cp in cps:
            cp.wait()

        outs = []
        for t in range(NSUB - 1):
            outs.append(pltpu.async_copy(
                stage_v.at[pl.ds(t * S, S)],
                out_hbm.at[b0 + k, pl.ds(0, S), pl.ds(t * 128, 128)],
                wsem))

        def row_body(s, c2):
            for c in range(6):
                slab_v[s, pl.ds(c * 16, 16)] = (
                    stage_v[7 * S + s, pl.ds(c * 16, 16)])
            slab_v[s, pl.ds(TAIL - 16, 16)] = (
                stage_v[7 * S + s, pl.ds(TAIL - 16, 16)])
            return c2

        lax.fori_loop(0, S, row_body, 0)
        outs.append(pltpu.async_copy(
            slab_v,
            out_hbm.at[b0 + k, pl.ds(0, S), pl.ds(7 * 128, TAIL)],
            wsem))
        for cp in outs:
            cp.wait()
        return carry

    lax.fori_loop(0, BPW, batch_body, 0)


def kernel(idx, w):
    w8 = jnp.pad(w, ((0, 0), (0, 24))).reshape(NSUB * 1000, 128)
    idx8 = (idx.astype(jnp.int32)[:, None, :] * NSUB
            + jnp.arange(NSUB, dtype=jnp.int32)[None, :, None]).reshape(-1)
    return _gather_kernel(idx8, w8)


# speedup vs baseline: 1.9581x; 1.0335x over previous
import functools

import jax
import jax.numpy as jnp
from jax import lax
from jax.experimental import pallas as pl
from jax.experimental.pallas import tpu as pltpu
from jax.experimental.pallas import tpu_sc as plsc

NB = 1024
S = 50
D = 1000
NSUB = 8
QB = S * NSUB
TAIL = D - 7 * 128
NC, NS_ = 2, 16
NW = NC * NS_
BPW = NB // NW
CHUNKS = ((0, 128), (128, 128), (256, 128), (384, QB - 384))

_mesh = plsc.VectorSubcoreMesh(core_axis_name="c", subcore_axis_name="s")


@functools.partial(
    pl.kernel,
    mesh=_mesh,
    out_type=jax.ShapeDtypeStruct((NB, S, D), jnp.float32),
    scratch_types=[
        pltpu.VMEM((BPW * QB,), jnp.int32),
        pltpu.VMEM((QB, 128), jnp.float32),
        pltpu.VMEM((QB, 128), jnp.float32),
        pltpu.VMEM((S, TAIL), jnp.float32),
        pltpu.VMEM((S, TAIL), jnp.float32),
        pltpu.SemaphoreType.DMA,
        pltpu.SemaphoreType.DMA,
    ],
)
def _gather_kernel(idx_hbm, w8_hbm, out_hbm, idx_v, stage_a, stage_b,
                   slab_a, slab_b, gsem, wsem):
    wid = lax.axis_index("s") * NC + lax.axis_index("c")
    b0 = wid * BPW
    pltpu.sync_copy(idx_hbm.at[pl.ds(b0 * QB, BPW * QB)], idx_v)

    def g_copies(k, stage):
        q0 = k * QB
        return [(w8_hbm.at[idx_v.at[pl.ds(q0 + j, n)]],
                 stage.at[pl.ds(j, n)]) for j, n in CHUNKS]

    def w_copies(k, stage, slab):
        b = b0 + k
        cps = [(stage.at[pl.ds(t * S, S)],
                out_hbm.at[b, pl.ds(0, S), pl.ds(t * 128, 128)])
               for t in range(NSUB - 1)]
        cps.append((slab, out_hbm.at[b, pl.ds(0, S), pl.ds(7 * 128, TAIL)]))
        return cps

    def issue(cps, sem):
        for src, dst in cps:
            pltpu.async_copy(src, dst, sem)

    def drain(cps, sem):
        for src, dst in cps:
            pltpu.make_async_copy(src, dst, sem).wait()

    def repack(stage, slab):
        def row_body(s, c2):
            for c in range(6):
                slab[s, pl.ds(c * 16, 16)] = (
                    stage[7 * S + s, pl.ds(c * 16, 16)])
            slab[s, pl.ds(TAIL - 16, 16)] = (
                stage[7 * S + s, pl.ds(TAIL - 16, 16)])
            return c2
        lax.fori_loop(0, S, row_body, 0)

    def step(k, cur, curslab, other, otherslab):
        drain(g_copies(k, cur), gsem)
        drain(w_copies(k - 1, other, otherslab), wsem)
        issue(g_copies(k + 1, other), gsem)
        repack(cur, curslab)
        issue(w_copies(k, cur, curslab), wsem)

    issue(g_copies(0, stage_a), gsem)
    drain(g_copies(0, stage_a), gsem)
    issue(g_copies(1, stage_b), gsem)
    repack(stage_a, slab_a)
    issue(w_copies(0, stage_a, slab_a), wsem)

    def mid(i, carry):
        k1 = 2 * i + 1
        step(k1, stage_b, slab_b, stage_a, slab_a)
        step(k1 + 1, stage_a, slab_a, stage_b, slab_b)
        return carry

    lax.fori_loop(0, (BPW - 2) // 2, mid, 0)

    k = BPW - 1
    drain(g_copies(k, stage_b), gsem)
    drain(w_copies(k - 1, stage_a, slab_a), wsem)
    repack(stage_b, slab_b)
    issue(w_copies(k, stage_b, slab_b), wsem)
    drain(w_copies(k, stage_b, slab_b), wsem)


def kernel(idx, w):
    w8 = jnp.pad(w, ((0, 0), (0, 24))).reshape(NSUB * 1000, 128)
    idx8 = (idx.astype(jnp.int32)[:, None, :] * NSUB
            + jnp.arange(NSUB, dtype=jnp.int32)[None, :, None]).reshape(-1)
    return _gather_kernel(idx8, w8)
